# baseline (speedup 1.0000x reference)
import functools

import jax
import jax.numpy as jnp
from jax import lax
from jax.experimental import pallas as pl
from jax.experimental.pallas import tpu as pltpu
from jax.experimental.pallas import tpu_sc as plsc

E = 800000
D = 64
NUM_CORES = 2
NUM_SUBCORES = 16
NW = NUM_CORES * NUM_SUBCORES
CP = 320
T = (E // 2) // CP
Q, R = divmod(T, NW)
MAXT = Q + 1
GROUPS = MAXT * CP // 16


@jax.jit
def _sc_embed(idx, ptab):
    mesh = plsc.VectorSubcoreMesh(core_axis_name="c", subcore_axis_name="s")

    @functools.partial(
        pl.kernel,
        mesh=mesh,
        out_type=jax.ShapeDtypeStruct((E // 2, 2 * D), jnp.float32),
        scratch_types=[
            pltpu.VMEM((MAXT * 2 * CP,), jnp.int32),
            pltpu.VMEM((MAXT * CP,), jnp.int32),
            pltpu.VMEM((2 * CP, 2 * D), jnp.float32),
            pltpu.SemaphoreType.DMA,
            pltpu.SemaphoreType.DMA,
            pltpu.SemaphoreType.DMA,
            pltpu.SemaphoreType.DMA,
        ],
        compiler_params=pltpu.CompilerParams(needs_layout_passes=False),
    )
    def k(idx_hbm, ptab_hbm, out_hbm, idx_v, pair_v, rows_v, g0, g1, w0, w1):
        wid = lax.axis_index("s") * NUM_CORES + lax.axis_index("c")
        start = jnp.minimum(wid * Q + jnp.minimum(wid, R), T - MAXT)

        pltpu.sync_copy(idx_hbm.at[pl.ds(start * 2 * CP, MAXT * 2 * CP)], idx_v)

        two_iota = lax.iota(jnp.int32, 16) * 2

        def pair_body(g, carry):
            pos = two_iota + g * 32
            ev = plsc.load_gather(idx_v, [pos])
            od = plsc.load_gather(idx_v, [pos + 1])
            pair_v[pl.ds(g * 16, 16)] = jnp.bitwise_and(ev * 4 + od, 15)
            return carry

        lax.fori_loop(0, GROUPS, pair_body, 0)

        gsem = (g0, g1)
        wsem = (w0, w1)

        def gather(ci, b):
            return pltpu.async_copy(
                ptab_hbm.at[pair_v.at[pl.ds(ci * CP, CP)]],
                rows_v.at[pl.ds(b * CP, CP)],
                gsem[b],
            )

        def write(ci, b):
            return pltpu.async_copy(
                rows_v.at[pl.ds(b * CP, CP)],
                out_hbm.at[pl.ds((start + ci) * CP, CP)],
                wsem[b],
            )

        g_desc = [gather(0, 0), None]
        w_desc = [None, None]
        for ci in range(MAXT):
            b = ci & 1
            g_desc[b].wait()
            if ci + 1 < MAXT:
                ob = 1 - b
                if w_desc[ob] is not None:
                    w_desc[ob].wait()
                g_desc[ob] = gather(ci + 1, ob)
            w_desc[b] = write(ci, b)
        w_desc[(MAXT - 1) & 1].wait()
        w_desc[(MAXT - 2) & 1].wait()

    return k(idx, ptab)


def kernel(edge_type, table):
    idx = edge_type.astype(jnp.int32)
    ptab = jnp.concatenate(
        [jnp.repeat(table, 4, axis=0), jnp.tile(table, (4, 1))], axis=1
    )
    out2 = _sc_embed(idx, ptab)
    return out2.reshape(E, D)


# speedup vs baseline: 2.2283x; 2.2283x over previous
import functools

import jax
import jax.numpy as jnp
from jax import lax
from jax.experimental import pallas as pl
from jax.experimental.pallas import tpu as pltpu
from jax.experimental.pallas import tpu_sc as plsc

E = 800000
D = 64
NUM_CORES = 2
NUM_SUBCORES = 16
NW = NUM_CORES * NUM_SUBCORES
CP = 320
T = (E // 2) // CP
Q, R = divmod(T, NW)
MAXT = Q + 1
GROUPS = MAXT * CP // 16


@jax.jit
def _sc_embed(idx, ptab):
    mesh = plsc.VectorSubcoreMesh(core_axis_name="c", subcore_axis_name="s")

    @functools.partial(
        pl.kernel,
        mesh=mesh,
        out_type=jax.ShapeDtypeStruct((E // 2, 2 * D), jnp.float32),
        scratch_types=[
            pltpu.VMEM((MAXT * 2 * CP,), jnp.int32),
            pltpu.VMEM((MAXT * CP,), jnp.int32),
            pltpu.VMEM((2 * CP, 2 * D), jnp.float32),
            pltpu.SemaphoreType.DMA,
            pltpu.SemaphoreType.DMA,
            pltpu.SemaphoreType.DMA,
            pltpu.SemaphoreType.DMA,
        ],
        compiler_params=pltpu.CompilerParams(needs_layout_passes=False),
    )
    def k(idx_hbm, ptab_hbm, out_hbm, idx_v, pair_v, rows_v, g0, g1, w0, w1):
        wid = lax.axis_index("s") * NUM_CORES + lax.axis_index("c")
        start = jnp.minimum(wid * Q + jnp.minimum(wid, R), T - MAXT)

        pltpu.sync_copy(idx_hbm.at[pl.ds(start * 2 * CP, MAXT * 2 * CP)], idx_v)

        two_iota = lax.iota(jnp.int32, 16) * 2

        def pair_body(g, carry):
            pos = two_iota + g * 32
            ev = plsc.load_gather(idx_v, [pos])
            od = plsc.load_gather(idx_v, [pos + 1])
            pair_v[pl.ds(g * 16, 16)] = (
                jnp.bitwise_and(ev * 4 + od, 15) + wid * 16
            )
            return carry

        lax.fori_loop(0, GROUPS, pair_body, 0)

        gsem = (g0, g1)
        wsem = (w0, w1)

        def gather(ci, b):
            return pltpu.async_copy(
                ptab_hbm.at[pair_v.at[pl.ds(ci * CP, CP)]],
                rows_v.at[pl.ds(b * CP, CP)],
                gsem[b],
            )

        def write(ci, b):
            return pltpu.async_copy(
                rows_v.at[pl.ds(b * CP, CP)],
                out_hbm.at[pl.ds((start + ci) * CP, CP)],
                wsem[b],
            )

        g_desc = [gather(0, 0), None]
        w_desc = [None, None]
        for ci in range(MAXT):
            b = ci & 1
            g_desc[b].wait()
            if ci + 1 < MAXT:
                ob = 1 - b
                if w_desc[ob] is not None:
                    w_desc[ob].wait()
                g_desc[ob] = gather(ci + 1, ob)
            w_desc[b] = write(ci, b)
        w_desc[(MAXT - 1) & 1].wait()
        w_desc[(MAXT - 2) & 1].wait()

    return k(idx, ptab)


def kernel(edge_type, table):
    idx = edge_type.astype(jnp.int32)
    ptab = jnp.concatenate(
        [jnp.repeat(table, 4, axis=0), jnp.tile(table, (4, 1))], axis=1
    )
    ptab = jnp.tile(ptab, (NW, 1))
    out2 = _sc_embed(idx, ptab)
    return out2.reshape(E, D)


# speedup vs baseline: 2.7181x; 1.2198x over previous
import functools

import jax
import jax.numpy as jnp
from jax import lax
from jax.experimental import pallas as pl
from jax.experimental.pallas import tpu as pltpu
from jax.experimental.pallas import tpu_sc as plsc

E = 800000
D = 64
NUM_CORES = 2
NUM_SUBCORES = 16
NW = NUM_CORES * NUM_SUBCORES
CP = 320
T = (E // 2) // CP
Q, R = divmod(T, NW)
MAXT = Q + 1
GROUPS = MAXT * CP // 16
NREP = 256


@jax.jit
def _sc_embed(idx, ptab):
    mesh = plsc.VectorSubcoreMesh(core_axis_name="c", subcore_axis_name="s")

    @functools.partial(
        pl.kernel,
        mesh=mesh,
        out_type=jax.ShapeDtypeStruct((E // 2, 2 * D), jnp.float32),
        scratch_types=[
            pltpu.VMEM((MAXT * 2 * CP,), jnp.int32),
            pltpu.VMEM((MAXT * CP,), jnp.int32),
            pltpu.VMEM((2 * CP, 2 * D), jnp.float32),
            pltpu.SemaphoreType.DMA,
            pltpu.SemaphoreType.DMA,
            pltpu.SemaphoreType.DMA,
            pltpu.SemaphoreType.DMA,
        ],
        compiler_params=pltpu.CompilerParams(needs_layout_passes=False),
    )
    def k(idx_hbm, ptab_hbm, out_hbm, idx_v, pair_v, rows_v, g0, g1, w0, w1):
        wid = lax.axis_index("s") * NUM_CORES + lax.axis_index("c")
        start = jnp.minimum(wid * Q + jnp.minimum(wid, R), T - MAXT)

        pltpu.sync_copy(idx_hbm.at[pl.ds(start * 2 * CP, MAXT * 2 * CP)], idx_v)

        two_iota = lax.iota(jnp.int32, 16) * 2

        def pair_body(g, carry):
            pos = two_iota + g * 32
            ev = plsc.load_gather(idx_v, [pos])
            od = plsc.load_gather(idx_v, [pos + 1])
            rep = jnp.bitwise_and(wid * GROUPS + g, NREP - 1)
            pair_v[pl.ds(g * 16, 16)] = (
                jnp.bitwise_and(ev * 4 + od, 15) + rep * 16
            )
            return carry

        lax.fori_loop(0, GROUPS, pair_body, 0)

        gsem = (g0, g1)
        wsem = (w0, w1)

        def gather(ci, b):
            return pltpu.async_copy(
                ptab_hbm.at[pair_v.at[pl.ds(ci * CP, CP)]],
                rows_v.at[pl.ds(b * CP, CP)],
                gsem[b],
            )

        def write(ci, b):
            return pltpu.async_copy(
                rows_v.at[pl.ds(b * CP, CP)],
                out_hbm.at[pl.ds((start + ci) * CP, CP)],
                wsem[b],
            )

        g_desc = [gather(0, 0), None]
        w_desc = [None, None]
        for ci in range(MAXT):
            b = ci & 1
            g_desc[b].wait()
            if ci + 1 < MAXT:
                ob = 1 - b
                if w_desc[ob] is not None:
                    w_desc[ob].wait()
                g_desc[ob] = gather(ci + 1, ob)
            w_desc[b] = write(ci, b)
        w_desc[(MAXT - 1) & 1].wait()
        w_desc[(MAXT - 2) & 1].wait()

    return k(idx, ptab)


def kernel(edge_type, table):
    idx = edge_type.astype(jnp.int32)
    ptab = jnp.concatenate(
        [jnp.repeat(table, 4, axis=0), jnp.tile(table, (4, 1))], axis=1
    )
    ptab = jnp.tile(ptab, (NREP, 1))
    out2 = _sc_embed(idx, ptab)
    return out2.reshape(E, D)
